# baseline (speedup 1.0000x reference)
import functools

import jax
import jax.numpy as jnp
from jax import lax
from jax.experimental import pallas as pl
from jax.experimental.pallas import tpu as pltpu
from jax.experimental.pallas import tpu_sc as plsc

_K = 512
_ROWS = 64
_N = 8192
_L = 16
_NB = 256
_CH = _N // _L
_RPW = 2
_INT_MIN = -(2 ** 31)


def _sc_body(x_hbm, out_hbm, xv, ov, ikeyv, subv, subcolv, histv):
    wid = lax.axis_index("s") * 2 + lax.axis_index("c")
    base_row = wid * _RPW
    pltpu.sync_copy(x_hbm.at[pl.ds(base_row, _RPW)], xv)
    lanes = lax.iota(jnp.int32, _L)
    ones = jnp.ones((_L,), jnp.int32)

    for rr in range(_RPW):
        def zero_hist(i, c):
            histv[pl.ds(i * _L, _L)] = jnp.zeros((_L,), jnp.int32)
            return c
        lax.fori_loop(0, _NB * _L // _L, zero_hist, 0)

        def histpass(i, c):
            v = xv[rr, pl.ds(i * _L, _L)]
            bits = plsc.bitcast(v, jnp.int32)
            ik = jnp.where(bits < 0, bits ^ jnp.int32(0x7FFFFFFF), bits)
            ikeyv[pl.ds(i * _L, _L)] = ik
            bucket = (ik >> 24) + 128
            plsc.addupdate_scatter(histv, [lanes * _NB + bucket], ones)
            return c
        lax.fori_loop(0, _CH, histpass, 0)

        carry = jnp.int32(0)
        m_enc = jnp.int32(-1)
        for v16 in reversed(range(_NB // _L)):
            tv = histv[pl.ds(v16 * _L, _L)]
            for l in range(1, _L):
                tv = tv + histv[pl.ds(l * _NB + v16 * _L, _L)]
            rv = lax.rev(tv, (0,))
            cs = plsc.cumsum(rv) + carry
            bv = jnp.int32(v16 * _L + 15) - lanes
            enc = jnp.where(cs >= _K, (bv << 14) | (cs - rv), jnp.int32(-1))
            m_enc = jnp.maximum(m_enc, jnp.max(enc))
            carry = carry + jnp.sum(tv)
        b_star = m_enc >> 14
        s_above = m_enc & jnp.int32(16383)
        r = _K - s_above

        def cpass(i, off):
            ik = ikeyv[pl.ds(i * _L, _L)]
            bucket = (ik >> 24) + 128
            msk = bucket == b_star
            cs = plsc.cumsum(msk.astype(jnp.int32))
            pos = off + cs - 1
            plsc.store_scatter(subv, [pos], ik, mask=msk)
            plsc.store_scatter(subcolv, [pos], lanes + i * _L, mask=msk)
            return off + jnp.max(cs)
        n_sub = lax.fori_loop(0, _CH, cpass, jnp.int32(0))
        plsc.store_scatter(subv, [n_sub + lanes],
                           jnp.full((_L,), _INT_MIN, jnp.int32))
        plsc.store_scatter(subcolv, [n_sub + lanes],
                           jnp.full((_L,), _N, jnp.int32))
        nch = (n_sub + _L - 1) // _L

        t = (b_star - 128) << 24
        for b in range(23, -1, -1):
            cand = t | jnp.int32(1 << b)
            def bitpass(i, acc):
                return acc + (subv[pl.ds(i * _L, _L)] >= cand).astype(jnp.int32)
            acc = lax.fori_loop(0, nch, bitpass, jnp.zeros((_L,), jnp.int32))
            t = jnp.where(jnp.sum(acc) >= r, cand, t)

        def gtpass(i, acc):
            return acc + (subv[pl.ds(i * _L, _L)] > t).astype(jnp.int32)
        acc = lax.fori_loop(0, nch, gtpass, jnp.zeros((_L,), jnp.int32))
        need = r - jnp.sum(acc)

        def jpass(i, carry2):
            pr, jacc = carry2
            kv = subv[pl.ds(i * _L, _L)]
            mk = kv == t
            cs = plsc.cumsum(mk.astype(jnp.int32)) + pr
            sel = mk & (cs == need)
            jv = jnp.where(sel, subcolv[pl.ds(i * _L, _L)], jnp.int32(-1))
            return jnp.max(cs), jnp.maximum(jacc, jv)
        _, jacc = lax.fori_loop(0, nch, jpass,
                                (jnp.int32(0), jnp.full((_L,), -1, jnp.int32)))
        jcut = jnp.max(jacc)

        def fpass(i, c):
            ik = ikeyv[pl.ds(i * _L, _L)]
            colv = lanes + i * _L
            keep = (ik > t) | ((ik == t) & (colv <= jcut))
            xvv = xv[rr, pl.ds(i * _L, _L)]
            ov[rr, pl.ds(i * _L, _L)] = jnp.where(
                keep, jnp.maximum(xvv, 0.0), 0.0)
            return c
        lax.fori_loop(0, _CH, fpass, 0)

    pltpu.sync_copy(ov, out_hbm.at[pl.ds(base_row, _RPW)])


@jax.jit
def _sc_topk(x):
    mesh = plsc.VectorSubcoreMesh(core_axis_name="c", subcore_axis_name="s")
    return pl.kernel(
        _sc_body,
        out_type=jax.ShapeDtypeStruct((_ROWS, _N), jnp.float32),
        mesh=mesh,
        compiler_params=pltpu.CompilerParams(needs_layout_passes=False),
        scratch_types=[
            pltpu.VMEM((_RPW, _N), jnp.float32),
            pltpu.VMEM((_RPW, _N), jnp.float32),
            pltpu.VMEM((_N,), jnp.int32),
            pltpu.VMEM((_N + _L,), jnp.int32),
            pltpu.VMEM((_N + _L,), jnp.int32),
            pltpu.VMEM((_NB * _L,), jnp.int32),
        ],
    )(x)


def kernel(x):
    return _sc_topk(x)


# speedup vs baseline: 1.4065x; 1.4065x over previous
import functools

import jax
import jax.numpy as jnp
from jax import lax
from jax.experimental import pallas as pl
from jax.experimental.pallas import tpu as pltpu
from jax.experimental.pallas import tpu_sc as plsc

_K = 512
_ROWS = 64
_N = 8192
_L = 16
_NB = 256
_CH = _N // _L
_RPW = 2
_UNROLL = 4
_INT_MIN = -(2 ** 31)


def _hist_select(histv, lanes, target_rank):
    carry = jnp.int32(0)
    m_enc = jnp.int32(-1)
    for v16 in reversed(range(_NB // _L)):
        tv = histv[pl.ds(v16 * _L, _L)]
        for l in range(1, _L):
            tv = tv + histv[pl.ds(l * _NB + v16 * _L, _L)]
        rv = lax.rev(tv, (0,))
        cs = plsc.cumsum(rv) + carry
        bv = jnp.int32(v16 * _L + 15) - lanes
        enc = jnp.where(cs >= target_rank, (bv << 14) | (cs - rv),
                        jnp.int32(-1))
        m_enc = jnp.maximum(m_enc, jnp.max(enc))
        carry = carry + jnp.sum(tv)
    b_star = m_enc >> 14
    rank_in = target_rank - (m_enc & jnp.int32(16383))
    return b_star, rank_in


def _sc_body(x_hbm, out_hbm, xv, ov, ikeyv, subv, subcolv, sub2v, sub2colv,
             histv):
    wid = lax.axis_index("s") * 2 + lax.axis_index("c")
    base_row = wid * _RPW
    pltpu.sync_copy(x_hbm.at[pl.ds(base_row, _RPW)], xv)
    lanes = lax.iota(jnp.int32, _L)
    lanes_nb = lanes * _NB
    ones = jnp.ones((_L,), jnp.int32)
    zeros = jnp.zeros((_L,), jnp.int32)

    for rr in range(_RPW):
        def zero_hist(i, c):
            for u in range(_UNROLL):
                histv[pl.ds((i * _UNROLL + u) * _L, _L)] = zeros
            return c
        lax.fori_loop(0, _NB * _L // _L // _UNROLL, zero_hist, 0)

        def histpass(i, c):
            for u in range(_UNROLL):
                s = (i * _UNROLL + u) * _L
                v = xv[rr, pl.ds(s, _L)]
                bits = plsc.bitcast(v, jnp.int32)
                ik = jnp.where(bits < 0, bits ^ jnp.int32(0x7FFFFFFF), bits)
                ikeyv[pl.ds(s, _L)] = ik
                bucket = (ik >> 24) + 128
                plsc.addupdate_scatter(histv, [lanes_nb + bucket], ones)
            return c
        lax.fori_loop(0, _CH // _UNROLL, histpass, 0)

        b_star, r1 = _hist_select(histv, lanes, jnp.int32(_K))

        def cpass(i, off):
            for u in range(_UNROLL):
                s = (i * _UNROLL + u) * _L
                ik = ikeyv[pl.ds(s, _L)]
                msk = ((ik >> 24) + 128) == b_star
                cs = plsc.cumsum(msk.astype(jnp.int32))
                pos = off + cs - 1
                plsc.store_scatter(subv, [pos], ik, mask=msk)
                plsc.store_scatter(subcolv, [pos], lanes + s, mask=msk)
                off = off + plsc.all_reduce_population_count(msk)
            return off
        off = lax.fori_loop(0, _CH // _UNROLL, cpass, zeros)
        n_sub = jnp.max(off)
        plsc.store_scatter(subv, [n_sub + lanes],
                           jnp.full((_L,), _INT_MIN, jnp.int32))
        plsc.store_scatter(subcolv, [n_sub + lanes],
                           jnp.full((_L,), _N, jnp.int32))
        nch = (n_sub + _L - 1) >> 4

        def zero_hist2(i, c):
            histv[pl.ds(i * _L, _L)] = zeros
            return c
        lax.fori_loop(0, _NB * _L // _L, zero_hist2, 0)

        def histpass2(i, c):
            ik = subv[pl.ds(i * _L, _L)]
            bucket = (ik >> 16) & jnp.int32(0xFF)
            plsc.addupdate_scatter(histv, [lanes_nb + bucket], ones)
            return c
        lax.fori_loop(0, nch, histpass2, 0)

        b2_star, r2 = _hist_select(histv, lanes, r1)

        def cpass2(i, off2):
            ik = subv[pl.ds(i * _L, _L)]
            msk = ((ik >> 16) & jnp.int32(0xFF)) == b2_star
            cs = plsc.cumsum(msk.astype(jnp.int32))
            pos = off2 + cs - 1
            plsc.store_scatter(sub2v, [pos], ik, mask=msk)
            plsc.store_scatter(sub2colv, [pos], subcolv[pl.ds(i * _L, _L)],
                               mask=msk)
            return off2 + plsc.all_reduce_population_count(msk)
        off2 = lax.fori_loop(0, nch, cpass2, zeros)
        n3 = jnp.max(off2)
        plsc.store_scatter(sub2v, [n3 + lanes],
                           jnp.full((_L,), _INT_MIN, jnp.int32))
        plsc.store_scatter(sub2colv, [n3 + lanes],
                           jnp.full((_L,), _N, jnp.int32))
        nch3 = (n3 + _L - 1) >> 4

        t = ((b_star - 128) << 24) | (b2_star << 16)
        for b in range(15, -1, -1):
            cand = t | jnp.int32(1 << b)

            def bitpass(i, acc):
                return acc + (sub2v[pl.ds(i * _L, _L)] >= cand).astype(
                    jnp.int32)
            acc = lax.fori_loop(0, nch3, bitpass, zeros)
            t = jnp.where(jnp.sum(acc) >= r2, cand, t)

        def gtpass(i, acc):
            return acc + (sub2v[pl.ds(i * _L, _L)] > t).astype(jnp.int32)
        acc = lax.fori_loop(0, nch3, gtpass, zeros)
        need = r2 - jnp.sum(acc)

        def jpass(i, carry2):
            pr, jacc = carry2
            kv = sub2v[pl.ds(i * _L, _L)]
            mk = kv == t
            cs = plsc.cumsum(mk.astype(jnp.int32)) + pr
            sel = mk & (cs == need)
            jv = jnp.where(sel, sub2colv[pl.ds(i * _L, _L)], jnp.int32(-1))
            return jnp.max(cs), jnp.maximum(jacc, jv)
        _, jacc = lax.fori_loop(0, nch3, jpass,
                                (jnp.int32(0), jnp.full((_L,), -1, jnp.int32)))
        jcut = jnp.max(jacc)

        def fpass(i, c):
            for u in range(_UNROLL):
                s = (i * _UNROLL + u) * _L
                ik = ikeyv[pl.ds(s, _L)]
                colv = lanes + s
                keep = (ik > t) | ((ik == t) & (colv <= jcut))
                xvv = xv[rr, pl.ds(s, _L)]
                ov[rr, pl.ds(s, _L)] = jnp.where(
                    keep, jnp.maximum(xvv, 0.0), 0.0)
            return c
        lax.fori_loop(0, _CH // _UNROLL, fpass, 0)

    pltpu.sync_copy(ov, out_hbm.at[pl.ds(base_row, _RPW)])


@jax.jit
def _sc_topk(x):
    mesh = plsc.VectorSubcoreMesh(core_axis_name="c", subcore_axis_name="s")
    return pl.kernel(
        _sc_body,
        out_type=jax.ShapeDtypeStruct((_ROWS, _N), jnp.float32),
        mesh=mesh,
        compiler_params=pltpu.CompilerParams(needs_layout_passes=False),
        scratch_types=[
            pltpu.VMEM((_RPW, _N), jnp.float32),
            pltpu.VMEM((_RPW, _N), jnp.float32),
            pltpu.VMEM((_N,), jnp.int32),
            pltpu.VMEM((_N + _L,), jnp.int32),
            pltpu.VMEM((_N + _L,), jnp.int32),
            pltpu.VMEM((_N + _L,), jnp.int32),
            pltpu.VMEM((_N + _L,), jnp.int32),
            pltpu.VMEM((_NB * _L,), jnp.int32),
        ],
    )(x)


def kernel(x):
    return _sc_topk(x)


# speedup vs baseline: 1.4336x; 1.0193x over previous
import functools

import jax
import jax.numpy as jnp
from jax import lax
from jax.experimental import pallas as pl
from jax.experimental.pallas import tpu as pltpu
from jax.experimental.pallas import tpu_sc as plsc

_K = 512
_ROWS = 64
_N = 8192
_L = 16
_NB = 256
_CH = _N // _L
_RPW = 2
_UNROLL = 4
_INT_MIN = -(2 ** 31)


def _hist_select(histv, target_rank):
    blk_tot = []
    for blk in range(_NB // _L):
        acc = histv[pl.ds(blk * _NB, _L)]
        for j in range(1, _L):
            acc = acc + histv[pl.ds(blk * _NB + j * _L, _L)]
        blk_tot.append(jnp.sum(acc))

    carry = jnp.int32(0)
    found = jnp.bool_(False)
    b_blk = jnp.int32(0)
    s_above = jnp.int32(0)
    for blk in reversed(range(_NB // _L)):
        suffix = carry + blk_tot[blk]
        qual = suffix >= target_rank
        sel = jnp.logical_and(jnp.logical_not(found), qual)
        b_blk = jnp.where(sel, jnp.int32(blk), b_blk)
        s_above = jnp.where(sel, carry, s_above)
        found = jnp.logical_or(found, qual)
        carry = suffix

    base = b_blk * _NB
    carry2 = s_above
    found2 = jnp.bool_(False)
    b_star = jnp.int32(0)
    s_above2 = jnp.int32(0)
    for j in reversed(range(_L)):
        tot_j = jnp.sum(histv[pl.ds(base + j * _L, _L)])
        suffix = carry2 + tot_j
        qual = suffix >= target_rank
        sel = jnp.logical_and(jnp.logical_not(found2), qual)
        b_star = jnp.where(sel, b_blk * _L + jnp.int32(j), b_star)
        s_above2 = jnp.where(sel, carry2, s_above2)
        found2 = jnp.logical_or(found2, qual)
        carry2 = suffix
    return b_star, target_rank - s_above2


def _sc_body(x_hbm, out_hbm, xv, ov, ikeyv, subv, subcolv, sub2v, sub2colv,
             histv):
    wid = lax.axis_index("s") * 2 + lax.axis_index("c")
    base_row = wid * _RPW
    pltpu.sync_copy(x_hbm.at[pl.ds(base_row, _RPW)], xv)
    lanes = lax.iota(jnp.int32, _L)
    ones = jnp.ones((_L,), jnp.int32)
    zeros = jnp.zeros((_L,), jnp.int32)

    for rr in range(_RPW):
        def zero_hist(i, c):
            for u in range(_UNROLL):
                histv[pl.ds((i * _UNROLL + u) * _L, _L)] = zeros
            return c
        lax.fori_loop(0, _NB * _L // _L // _UNROLL, zero_hist, 0)

        def histpass(i, c):
            for u in range(_UNROLL):
                s = (i * _UNROLL + u) * _L
                v = xv[rr, pl.ds(s, _L)]
                bits = plsc.bitcast(v, jnp.int32)
                ik = jnp.where(bits < 0, bits ^ jnp.int32(0x7FFFFFFF), bits)
                ikeyv[pl.ds(s, _L)] = ik
                bucket = (ik >> 24) + 128
                plsc.addupdate_scatter(histv, [(bucket << 4) + lanes], ones)
            return c
        lax.fori_loop(0, _CH // _UNROLL, histpass, 0)

        b_star, r1 = _hist_select(histv, jnp.int32(_K))

        def cpass(i, off):
            for u in range(_UNROLL):
                s = (i * _UNROLL + u) * _L
                ik = ikeyv[pl.ds(s, _L)]
                msk = ((ik >> 24) + 128) == b_star
                cs = plsc.cumsum(msk.astype(jnp.int32))
                pos = off + cs - 1
                plsc.store_scatter(subv, [pos], ik, mask=msk)
                plsc.store_scatter(subcolv, [pos], lanes + s, mask=msk)
                off = off + plsc.all_reduce_population_count(msk)
            return off
        off = lax.fori_loop(0, _CH // _UNROLL, cpass, zeros)
        n_sub = jnp.max(off)
        plsc.store_scatter(subv, [n_sub + lanes],
                           jnp.full((_L,), _INT_MIN, jnp.int32))
        plsc.store_scatter(subcolv, [n_sub + lanes],
                           jnp.full((_L,), _N, jnp.int32))
        nch = (n_sub + _L - 1) >> 4

        def zero_hist2(i, c):
            for u in range(_UNROLL):
                histv[pl.ds((i * _UNROLL + u) * _L, _L)] = zeros
            return c
        lax.fori_loop(0, _NB * _L // _L // _UNROLL, zero_hist2, 0)

        def histpass2(i, c):
            ik = subv[pl.ds(i * _L, _L)]
            bucket = (ik >> 16) & jnp.int32(0xFF)
            plsc.addupdate_scatter(histv, [(bucket << 4) + lanes], ones)
            return c
        lax.fori_loop(0, nch, histpass2, 0)

        b2_star, r2 = _hist_select(histv, r1)

        def cpass2(i, off2):
            ik = subv[pl.ds(i * _L, _L)]
            msk = ((ik >> 16) & jnp.int32(0xFF)) == b2_star
            cs = plsc.cumsum(msk.astype(jnp.int32))
            pos = off2 + cs - 1
            plsc.store_scatter(sub2v, [pos], ik, mask=msk)
            plsc.store_scatter(sub2colv, [pos], subcolv[pl.ds(i * _L, _L)],
                               mask=msk)
            return off2 + plsc.all_reduce_population_count(msk)
        off2 = lax.fori_loop(0, nch, cpass2, zeros)
        n3 = jnp.max(off2)
        plsc.store_scatter(sub2v, [n3 + lanes],
                           jnp.full((_L,), _INT_MIN, jnp.int32))
        plsc.store_scatter(sub2colv, [n3 + lanes],
                           jnp.full((_L,), _N, jnp.int32))
        nch3 = (n3 + _L - 1) >> 4

        t = ((b_star - 128) << 24) | (b2_star << 16)
        for b in range(15, -1, -1):
            cand = t | jnp.int32(1 << b)

            def bitpass(i, acc):
                return acc + (sub2v[pl.ds(i * _L, _L)] >= cand).astype(
                    jnp.int32)
            acc = lax.fori_loop(0, nch3, bitpass, zeros)
            t = jnp.where(jnp.sum(acc) >= r2, cand, t)

        def gtpass(i, acc):
            return acc + (sub2v[pl.ds(i * _L, _L)] > t).astype(jnp.int32)
        acc = lax.fori_loop(0, nch3, gtpass, zeros)
        need = r2 - jnp.sum(acc)

        def jpass(i, carry2):
            pr, jacc = carry2
            kv = sub2v[pl.ds(i * _L, _L)]
            mk = kv == t
            cs = plsc.cumsum(mk.astype(jnp.int32)) + pr
            sel = mk & (cs == need)
            jv = jnp.where(sel, sub2colv[pl.ds(i * _L, _L)], jnp.int32(-1))
            return jnp.max(cs), jnp.maximum(jacc, jv)
        _, jacc = lax.fori_loop(0, nch3, jpass,
                                (jnp.int32(0), jnp.full((_L,), -1, jnp.int32)))
        jcut = jnp.max(jacc)

        def fpass(i, c):
            for u in range(_UNROLL):
                s = (i * _UNROLL + u) * _L
                ik = ikeyv[pl.ds(s, _L)]
                colv = lanes + s
                keep = (ik > t) | ((ik == t) & (colv <= jcut))
                xvv = xv[rr, pl.ds(s, _L)]
                ov[rr, pl.ds(s, _L)] = jnp.where(
                    keep, jnp.maximum(xvv, 0.0), 0.0)
            return c
        lax.fori_loop(0, _CH // _UNROLL, fpass, 0)

    pltpu.sync_copy(ov, out_hbm.at[pl.ds(base_row, _RPW)])


@jax.jit
def _sc_topk(x):
    mesh = plsc.VectorSubcoreMesh(core_axis_name="c", subcore_axis_name="s")
    return pl.kernel(
        _sc_body,
        out_type=jax.ShapeDtypeStruct((_ROWS, _N), jnp.float32),
        mesh=mesh,
        compiler_params=pltpu.CompilerParams(needs_layout_passes=False),
        scratch_types=[
            pltpu.VMEM((_RPW, _N), jnp.float32),
            pltpu.VMEM((_RPW, _N), jnp.float32),
            pltpu.VMEM((_N,), jnp.int32),
            pltpu.VMEM((_N + _L,), jnp.int32),
            pltpu.VMEM((_N + _L,), jnp.int32),
            pltpu.VMEM((_N + _L,), jnp.int32),
            pltpu.VMEM((_N + _L,), jnp.int32),
            pltpu.VMEM((_NB * _L,), jnp.int32),
        ],
    )(x)


def kernel(x):
    return _sc_topk(x)


# speedup vs baseline: 2.5368x; 1.7695x over previous
import functools

import jax
import jax.numpy as jnp
from jax import lax
from jax.experimental import pallas as pl
from jax.experimental.pallas import tpu as pltpu
from jax.experimental.pallas import tpu_sc as plsc

_K = 512
_ROWS = 64
_N = 8192
_L = 16
_NB = 256
_CH = _N // _L
_RPW = 2
_UNROLL = 4
_INT_MIN = -(2 ** 31)


def _hist_select(histv, target_rank):
    blk_tot = []
    for blk in range(_NB // _L):
        acc = histv[pl.ds(blk * _NB, _L)]
        for j in range(1, _L):
            acc = acc + histv[pl.ds(blk * _NB + j * _L, _L)]
        blk_tot.append(jnp.sum(acc))

    carry = jnp.int32(0)
    found = jnp.bool_(False)
    b_blk = jnp.int32(0)
    s_above = jnp.int32(0)
    for blk in reversed(range(_NB // _L)):
        suffix = carry + blk_tot[blk]
        qual = suffix >= target_rank
        sel = jnp.logical_and(jnp.logical_not(found), qual)
        b_blk = jnp.where(sel, jnp.int32(blk), b_blk)
        s_above = jnp.where(sel, carry, s_above)
        found = jnp.logical_or(found, qual)
        carry = suffix

    base = b_blk * _NB
    carry2 = s_above
    found2 = jnp.bool_(False)
    b_star = jnp.int32(0)
    s_above2 = jnp.int32(0)
    for j in reversed(range(_L)):
        tot_j = jnp.sum(histv[pl.ds(base + j * _L, _L)])
        suffix = carry2 + tot_j
        qual = suffix >= target_rank
        sel = jnp.logical_and(jnp.logical_not(found2), qual)
        b_star = jnp.where(sel, b_blk * _L + jnp.int32(j), b_star)
        s_above2 = jnp.where(sel, carry2, s_above2)
        found2 = jnp.logical_or(found2, qual)
        carry2 = suffix
    return b_star, target_rank - s_above2


def _sc_body(x_hbm, out_hbm, xv, ov, ikeyv, subv, subcolv, sub2v, sub2colv,
             histv):
    wid = lax.axis_index("s") * 2 + lax.axis_index("c")
    base_row = wid * _RPW
    pltpu.sync_copy(x_hbm.at[pl.ds(base_row, _RPW)], xv)
    lanes = lax.iota(jnp.int32, _L)
    ones = jnp.ones((_L,), jnp.int32)
    zeros = jnp.zeros((_L,), jnp.int32)

    for rr in range(_RPW):
        @plsc.parallel_loop(0, _NB, 1, unroll=8)
        def _(i):
            histv[pl.ds(i * _L, _L)] = zeros

        @plsc.parallel_loop(0, _CH, 1, unroll=8)
        def _(i):
            s = i * _L
            v = xv[rr, pl.ds(s, _L)]
            bits = plsc.bitcast(v, jnp.int32)
            ik = jnp.where(bits < 0, bits ^ jnp.int32(0x7FFFFFFF), bits)
            ikeyv[pl.ds(s, _L)] = ik
            bucket = (ik >> 24) + 128
            plsc.addupdate_scatter(histv, [(bucket << 4) + lanes], ones)

        b_star, r1 = _hist_select(histv, jnp.int32(_K))

        @plsc.parallel_loop(0, _CH, 1, unroll=4, carry=zeros)
        def off(i, off):
            s = i * _L
            ik = ikeyv[pl.ds(s, _L)]
            msk = ((ik >> 24) + 128) == b_star
            cs = plsc.cumsum(msk.astype(jnp.int32))
            pos = off + cs - 1
            plsc.store_scatter(subv, [pos], ik, mask=msk)
            plsc.store_scatter(subcolv, [pos], lanes + s, mask=msk)
            return off + plsc.all_reduce_population_count(msk)
        n_sub = jnp.max(off)
        plsc.store_scatter(subv, [n_sub + lanes],
                           jnp.full((_L,), _INT_MIN, jnp.int32))
        plsc.store_scatter(subcolv, [n_sub + lanes],
                           jnp.full((_L,), _N, jnp.int32))
        nch = (n_sub + _L - 1) >> 4

        @plsc.parallel_loop(0, _NB, 1, unroll=8)
        def _(i):
            histv[pl.ds(i * _L, _L)] = zeros

        @plsc.parallel_loop(0, nch, 1, unroll=4)
        def _(i):
            ik = subv[pl.ds(i * _L, _L)]
            bucket = (ik >> 16) & jnp.int32(0xFF)
            plsc.addupdate_scatter(histv, [(bucket << 4) + lanes], ones)

        b2_star, r2 = _hist_select(histv, r1)

        @plsc.parallel_loop(0, nch, 1, unroll=4, carry=zeros)
        def off2(i, off2):
            ik = subv[pl.ds(i * _L, _L)]
            msk = ((ik >> 16) & jnp.int32(0xFF)) == b2_star
            cs = plsc.cumsum(msk.astype(jnp.int32))
            pos = off2 + cs - 1
            plsc.store_scatter(sub2v, [pos], ik, mask=msk)
            plsc.store_scatter(sub2colv, [pos], subcolv[pl.ds(i * _L, _L)],
                               mask=msk)
            return off2 + plsc.all_reduce_population_count(msk)
        n3 = jnp.max(off2)
        plsc.store_scatter(sub2v, [n3 + lanes],
                           jnp.full((_L,), _INT_MIN, jnp.int32))
        plsc.store_scatter(sub2colv, [n3 + lanes],
                           jnp.full((_L,), _N, jnp.int32))
        nch3 = (n3 + _L - 1) >> 4

        t = ((b_star - 128) << 24) | (b2_star << 16)
        for b in range(15, -1, -1):
            cand = t | jnp.int32(1 << b)

            def bitpass(i, acc):
                return acc + (sub2v[pl.ds(i * _L, _L)] >= cand).astype(
                    jnp.int32)
            acc = lax.fori_loop(0, nch3, bitpass, zeros)
            t = jnp.where(jnp.sum(acc) >= r2, cand, t)

        def gtpass(i, acc):
            return acc + (sub2v[pl.ds(i * _L, _L)] > t).astype(jnp.int32)
        acc = lax.fori_loop(0, nch3, gtpass, zeros)
        need = r2 - jnp.sum(acc)

        def jpass(i, carry2):
            pr, jacc = carry2
            kv = sub2v[pl.ds(i * _L, _L)]
            mk = kv == t
            cs = plsc.cumsum(mk.astype(jnp.int32)) + pr
            sel = mk & (cs == need)
            jv = jnp.where(sel, sub2colv[pl.ds(i * _L, _L)], jnp.int32(-1))
            return jnp.max(cs), jnp.maximum(jacc, jv)
        _, jacc = lax.fori_loop(0, nch3, jpass,
                                (jnp.int32(0), jnp.full((_L,), -1, jnp.int32)))
        jcut = jnp.max(jacc)

        @plsc.parallel_loop(0, _CH, 1, unroll=8)
        def _(i):
            s = i * _L
            ik = ikeyv[pl.ds(s, _L)]
            colv = lanes + s
            keep = (ik > t) | ((ik == t) & (colv <= jcut))
            xvv = xv[rr, pl.ds(s, _L)]
            ov[rr, pl.ds(s, _L)] = jnp.where(keep, jnp.maximum(xvv, 0.0), 0.0)

    pltpu.sync_copy(ov, out_hbm.at[pl.ds(base_row, _RPW)])


@jax.jit
def _sc_topk(x):
    mesh = plsc.VectorSubcoreMesh(core_axis_name="c", subcore_axis_name="s")
    return pl.kernel(
        _sc_body,
        out_type=jax.ShapeDtypeStruct((_ROWS, _N), jnp.float32),
        mesh=mesh,
        compiler_params=pltpu.CompilerParams(needs_layout_passes=False),
        scratch_types=[
            pltpu.VMEM((_RPW, _N), jnp.float32),
            pltpu.VMEM((_RPW, _N), jnp.float32),
            pltpu.VMEM((_N,), jnp.int32),
            pltpu.VMEM((_N + _L,), jnp.int32),
            pltpu.VMEM((_N + _L,), jnp.int32),
            pltpu.VMEM((_N + _L,), jnp.int32),
            pltpu.VMEM((_N + _L,), jnp.int32),
            pltpu.VMEM((_NB * _L,), jnp.int32),
        ],
    )(x)


def kernel(x):
    return _sc_topk(x)


# speedup vs baseline: 2.6236x; 1.0342x over previous
import functools

import jax
import jax.numpy as jnp
from jax import lax
from jax.experimental import pallas as pl
from jax.experimental.pallas import tpu as pltpu
from jax.experimental.pallas import tpu_sc as plsc

_K = 512
_ROWS = 64
_N = 8192
_L = 16
_NB = 256
_CH = _N // _L
_RPW = 2
_UNROLL = 4
_INT_MIN = -(2 ** 31)


def _hist_select(histv, target_rank):
    blk_tot = []
    for blk in range(_NB // _L):
        acc = histv[pl.ds(blk * _NB, _L)]
        for j in range(1, _L):
            acc = acc + histv[pl.ds(blk * _NB + j * _L, _L)]
        blk_tot.append(jnp.sum(acc))

    carry = jnp.int32(0)
    found = jnp.bool_(False)
    b_blk = jnp.int32(0)
    s_above = jnp.int32(0)
    for blk in reversed(range(_NB // _L)):
        suffix = carry + blk_tot[blk]
        qual = suffix >= target_rank
        sel = jnp.logical_and(jnp.logical_not(found), qual)
        b_blk = jnp.where(sel, jnp.int32(blk), b_blk)
        s_above = jnp.where(sel, carry, s_above)
        found = jnp.logical_or(found, qual)
        carry = suffix

    base = b_blk * _NB
    carry2 = s_above
    found2 = jnp.bool_(False)
    b_star = jnp.int32(0)
    s_above2 = jnp.int32(0)
    for j in reversed(range(_L)):
        tot_j = jnp.sum(histv[pl.ds(base + j * _L, _L)])
        suffix = carry2 + tot_j
        qual = suffix >= target_rank
        sel = jnp.logical_and(jnp.logical_not(found2), qual)
        b_star = jnp.where(sel, b_blk * _L + jnp.int32(j), b_star)
        s_above2 = jnp.where(sel, carry2, s_above2)
        found2 = jnp.logical_or(found2, qual)
        carry2 = suffix
    return b_star, target_rank - s_above2


def _sc_body(x_hbm, out_hbm, xv, ov, ikeyv, subv, subcolv, sub2v, sub2colv,
             histv):
    wid = lax.axis_index("s") * 2 + lax.axis_index("c")
    base_row = wid * _RPW
    pltpu.sync_copy(x_hbm.at[pl.ds(base_row, _RPW)], xv)
    lanes = lax.iota(jnp.int32, _L)
    ones = jnp.ones((_L,), jnp.int32)
    zeros = jnp.zeros((_L,), jnp.int32)

    for rr in range(_RPW):
        @plsc.parallel_loop(0, _NB, 1, unroll=8)
        def _(i):
            histv[pl.ds(i * _L, _L)] = zeros

        @plsc.parallel_loop(0, _CH, 1, unroll=8)
        def _(i):
            s = i * _L
            v = xv[rr, pl.ds(s, _L)]
            bits = plsc.bitcast(v, jnp.int32)
            ik = jnp.where(bits < 0, bits ^ jnp.int32(0x7FFFFFFF), bits)
            ikeyv[pl.ds(s, _L)] = ik
            bucket = (ik >> 24) + 128
            plsc.addupdate_scatter(histv, [(bucket << 4) + lanes], ones)

        b_star, r1 = _hist_select(histv, jnp.int32(_K))

        @plsc.parallel_loop(0, _CH, 1, unroll=8, carry=zeros)
        def off(i, off):
            s = i * _L
            ik = ikeyv[pl.ds(s, _L)]
            msk = ((ik >> 24) + 128) == b_star
            cs = plsc.cumsum(msk.astype(jnp.int32))
            pos = off + cs - 1
            plsc.store_scatter(subv, [pos], ik, mask=msk)
            plsc.store_scatter(subcolv, [pos], lanes + s, mask=msk)
            return off + plsc.all_reduce_population_count(msk)
        n_sub = jnp.max(off)
        plsc.store_scatter(subv, [n_sub + lanes],
                           jnp.full((_L,), _INT_MIN, jnp.int32))
        plsc.store_scatter(subcolv, [n_sub + lanes],
                           jnp.full((_L,), _N, jnp.int32))
        nch = (n_sub + _L - 1) >> 4

        @plsc.parallel_loop(0, _NB, 1, unroll=8)
        def _(i):
            histv[pl.ds(i * _L, _L)] = zeros

        @plsc.parallel_loop(0, nch, 1, unroll=4)
        def _(i):
            ik = subv[pl.ds(i * _L, _L)]
            bucket = (ik >> 16) & jnp.int32(0xFF)
            plsc.addupdate_scatter(histv, [(bucket << 4) + lanes], ones)

        b2_star, r2 = _hist_select(histv, r1)

        @plsc.parallel_loop(0, nch, 1, unroll=4, carry=zeros)
        def off2(i, off2):
            ik = subv[pl.ds(i * _L, _L)]
            msk = ((ik >> 16) & jnp.int32(0xFF)) == b2_star
            cs = plsc.cumsum(msk.astype(jnp.int32))
            pos = off2 + cs - 1
            plsc.store_scatter(sub2v, [pos], ik, mask=msk)
            plsc.store_scatter(sub2colv, [pos], subcolv[pl.ds(i * _L, _L)],
                               mask=msk)
            return off2 + plsc.all_reduce_population_count(msk)
        n3 = jnp.max(off2)
        plsc.store_scatter(sub2v, [n3 + lanes],
                           jnp.full((_L,), _INT_MIN, jnp.int32))
        plsc.store_scatter(sub2colv, [n3 + lanes],
                           jnp.full((_L,), _N, jnp.int32))
        nch3 = (n3 + _L - 1) >> 4

        r2v = jnp.broadcast_to(r2, (_L,))
        t = jnp.broadcast_to(((b_star - 128) << 24) | (b2_star << 16), (_L,))
        for b in range(15, -1, -1):
            cand = t | jnp.int32(1 << b)

            def bitpass(i, acc):
                m = sub2v[pl.ds(i * _L, _L)] >= cand
                return acc + plsc.all_reduce_population_count(m)
            acc = lax.fori_loop(0, nch3, bitpass, zeros)
            t = jnp.where(acc >= r2v, cand, t)

        def gtpass(i, acc):
            m = sub2v[pl.ds(i * _L, _L)] > t
            return acc + plsc.all_reduce_population_count(m)
        acc = lax.fori_loop(0, nch3, gtpass, zeros)
        need = r2v - acc

        def jpass(i, carry2):
            pr, jacc = carry2
            kv = sub2v[pl.ds(i * _L, _L)]
            mk = kv == t
            cs = plsc.cumsum(mk.astype(jnp.int32)) + pr
            sel = mk & (cs == need)
            jv = jnp.where(sel, sub2colv[pl.ds(i * _L, _L)], jnp.int32(-1))
            return pr + plsc.all_reduce_population_count(mk), \
                jnp.maximum(jacc, jv)
        _, jacc = lax.fori_loop(0, nch3, jpass,
                                (zeros, jnp.full((_L,), -1, jnp.int32)))
        jcut = jnp.max(jacc)

        @plsc.parallel_loop(0, _CH, 1, unroll=8)
        def _(i):
            s = i * _L
            ik = ikeyv[pl.ds(s, _L)]
            colv = lanes + s
            keep = (ik > t) | ((ik == t) & (colv <= jcut))
            xvv = xv[rr, pl.ds(s, _L)]
            ov[rr, pl.ds(s, _L)] = jnp.where(keep, jnp.maximum(xvv, 0.0), 0.0)

    pltpu.sync_copy(ov, out_hbm.at[pl.ds(base_row, _RPW)])


@jax.jit
def _sc_topk(x):
    mesh = plsc.VectorSubcoreMesh(core_axis_name="c", subcore_axis_name="s")
    return pl.kernel(
        _sc_body,
        out_type=jax.ShapeDtypeStruct((_ROWS, _N), jnp.float32),
        mesh=mesh,
        compiler_params=pltpu.CompilerParams(needs_layout_passes=False),
        scratch_types=[
            pltpu.VMEM((_RPW, _N), jnp.float32),
            pltpu.VMEM((_RPW, _N), jnp.float32),
            pltpu.VMEM((_N,), jnp.int32),
            pltpu.VMEM((_N + _L,), jnp.int32),
            pltpu.VMEM((_N + _L,), jnp.int32),
            pltpu.VMEM((_N + _L,), jnp.int32),
            pltpu.VMEM((_N + _L,), jnp.int32),
            pltpu.VMEM((_NB * _L,), jnp.int32),
        ],
    )(x)


def kernel(x):
    return _sc_topk(x)
